# baseline (device time: 70757 ns/iter reference)
import jax
import jax.numpy as jnp
from jax import lax
from jax.experimental import pallas as pl
from jax.experimental.pallas import tpu as pltpu

N_DEV = 16
KI = 2
NO = 8
GRP = N_DEV // KI
R = 4
NSTEP = KI * NO


def kernel(x, w_mat):
    m_total, k_shard = x.shape
    k_total, n_total = w_mat.shape
    m_blk = m_total // N_DEV
    kb = k_total // KI
    nb = n_total // NO

    def w_copy(w_hbm, wv_ref, w_sems, perm_ref, step):
        kg = step // NO
        c = lax.rem(step, NO)
        slot = lax.rem(step, R)
        return pltpu.make_async_copy(
            w_hbm.at[pl.ds(perm_ref[kg] * kb, kb), pl.ds(c * nb, nb)],
            wv_ref.at[slot],
            w_sems.at[slot],
        )

    def body(perm_ref, x_ref, w_hbm, out_ref, xbf_ref, xg_ref, xrow_ref,
             wv_ref, send_sems, recv_sems, w_sems):
        t = pl.program_id(0)
        kg = t // NO
        c = lax.rem(t, NO)
        my = lax.axis_index("i")
        g_my = my // GRP
        my_in = lax.rem(my, GRP)

        @pl.when(t == 0)
        def _():
            for s in range(R):
                w_copy(w_hbm, wv_ref, w_sems, perm_ref, s).start()
            xbf_ref[...] = x_ref[...].astype(jnp.bfloat16)
            xrow_ref[:, pl.ds(my * m_blk, m_blk)] = (
                xbf_ref[pl.ds(my * m_blk, m_blk), :]
            )

        dests = []
        for r in range(1, GRP):
            dests.append(g_my * GRP + lax.rem(my_in + r, GRP))
        for r in range(GRP):
            dests.append((1 - g_my) * GRP + lax.rem(my_in + r, GRP))

        for s, j in enumerate(dests):
            rdma = pltpu.make_async_remote_copy(
                src_ref=xbf_ref.at[pl.ds(j * m_blk, m_blk), :],
                dst_ref=xg_ref.at[my],
                send_sem=send_sems.at[s],
                recv_sem=recv_sems.at[my],
                device_id=j,
                device_id_type=pl.DeviceIdType.LOGICAL,
            )

            @pl.when(t == 0)
            def _():
                rdma.start()

            @pl.when(t == NSTEP - 1)
            def _():
                rdma.wait_send()

        g = perm_ref[kg]
        for r in range(GRP):
            o = g * GRP + r
            recv = pltpu.make_async_remote_copy(
                src_ref=xg_ref.at[o],
                dst_ref=xg_ref.at[o],
                send_sem=send_sems.at[0],
                recv_sem=recv_sems.at[o],
                device_id=my,
                device_id_type=pl.DeviceIdType.LOGICAL,
            )

            @pl.when((c == 0) & (o != my))
            def _():
                recv.wait_recv()
                xrow_ref[:, pl.ds(o * m_blk, m_blk)] = xg_ref[o]

        w_copy(w_hbm, wv_ref, w_sems, perm_ref, t).wait()
        prod = jnp.dot(
            xrow_ref[:, pl.ds(g * kb, kb)],
            wv_ref[lax.rem(t, R)].astype(jnp.bfloat16),
            preferred_element_type=jnp.float32,
        )
        sl = pl.ds(c * nb, nb)

        @pl.when(kg == 0)
        def _():
            out_ref[:, sl] = prod

        @pl.when(kg == KI - 1)
        def _():
            out_ref[:, sl] = jnp.maximum(out_ref[:, sl] + prod, 0.0)

        @pl.when(t + R < NSTEP)
        def _():
            w_copy(w_hbm, wv_ref, w_sems, perm_ref, t + R).start()

    grid_spec = pltpu.PrefetchScalarGridSpec(
        num_scalar_prefetch=1,
        grid=(NSTEP,),
        in_specs=[
            pl.BlockSpec((m_total, k_shard), lambda t, perm: (0, 0)),
            pl.BlockSpec(memory_space=pl.ANY),
        ],
        out_specs=pl.BlockSpec((m_blk, n_total), lambda t, perm: (0, 0)),
        scratch_shapes=[
            pltpu.VMEM((m_total, k_shard), jnp.bfloat16),
            pltpu.VMEM((N_DEV, m_blk, k_shard), jnp.bfloat16),
            pltpu.VMEM((m_blk, k_total), jnp.bfloat16),
            pltpu.VMEM((R, kb, nb), jnp.float32),
            pltpu.SemaphoreType.DMA((N_DEV,)),
            pltpu.SemaphoreType.DMA((N_DEV,)),
            pltpu.SemaphoreType.DMA((R,)),
        ],
    )

    g_my = lax.axis_index("i") // GRP
    perm = jnp.stack([g_my, 1 - g_my]).astype(jnp.int32)

    return pl.pallas_call(
        body,
        grid_spec=grid_spec,
        out_shape=jax.ShapeDtypeStruct((m_blk, n_total), jnp.float32),
        compiler_params=pltpu.CompilerParams(
            dimension_semantics=("arbitrary",),
            vmem_limit_bytes=60 * 1024 * 1024,
        ),
    )(perm, x, w_mat)


# device time: 66556 ns/iter; 1.0631x vs baseline; 1.0631x over previous
import jax
import jax.numpy as jnp
from jax import lax
from jax.experimental import pallas as pl
from jax.experimental.pallas import tpu as pltpu

N_DEV = 16
R = 4


def kernel(x, w_mat):
    m_total, k_shard = x.shape
    k_total, n_total = w_mat.shape
    m_blk = m_total // N_DEV

    def w_copy(w_hbm, wv_ref, w_sems, idx_ref, step):
        slot = lax.rem(step, R)
        return pltpu.make_async_copy(
            w_hbm.at[pl.ds(idx_ref[step] * k_shard, k_shard), :],
            wv_ref.at[slot],
            w_sems.at[slot],
        )

    def body(idx_ref, x_ref, w_hbm, out_ref, xbf_ref, xg_ref, wv_ref,
             send_sems, recv_sems, w_sems):
        t = pl.program_id(0)
        my = idx_ref[0]
        o = idx_ref[t]

        @pl.when(t == 0)
        def _():
            for s in range(R):
                w_copy(w_hbm, wv_ref, w_sems, idx_ref, s).start()
            xbf_ref[...] = x_ref[...].astype(jnp.bfloat16)
            xg_ref[my] = xbf_ref[pl.ds(my * m_blk, m_blk), :]
            barrier_sem = pltpu.get_barrier_semaphore()
            for p in range(N_DEV):
                pl.semaphore_signal(
                    barrier_sem, inc=1,
                    device_id=(p,), device_id_type=pl.DeviceIdType.MESH,
                )
            pl.semaphore_wait(barrier_sem, N_DEV)

        for s in range(1, N_DEV):
            j = lax.rem(my - s + N_DEV, N_DEV)
            rdma = pltpu.make_async_remote_copy(
                src_ref=xbf_ref.at[pl.ds(j * m_blk, m_blk), :],
                dst_ref=xg_ref.at[my],
                send_sem=send_sems.at[s],
                recv_sem=recv_sems.at[my],
                device_id=j,
                device_id_type=pl.DeviceIdType.LOGICAL,
            )

            @pl.when(t == 0)
            def _():
                rdma.start()

            @pl.when(t == N_DEV - 1)
            def _():
                rdma.wait_send()

        recv = pltpu.make_async_remote_copy(
            src_ref=xg_ref.at[o],
            dst_ref=xg_ref.at[o],
            send_sem=send_sems.at[0],
            recv_sem=recv_sems.at[o],
            device_id=my,
            device_id_type=pl.DeviceIdType.LOGICAL,
        )

        @pl.when(t != 0)
        def _():
            recv.wait_recv()

        w_copy(w_hbm, wv_ref, w_sems, idx_ref, t).wait()
        prod = jnp.dot(
            xg_ref[o],
            wv_ref[lax.rem(t, R)].astype(jnp.bfloat16),
            preferred_element_type=jnp.float32,
        )

        @pl.when(t == 0)
        def _():
            out_ref[...] = prod

        @pl.when((t != 0) & (t != N_DEV - 1))
        def _():
            out_ref[...] = out_ref[...] + prod

        @pl.when(t == N_DEV - 1)
        def _():
            out_ref[...] = jnp.maximum(out_ref[...] + prod, 0.0)

        @pl.when(t + R < N_DEV)
        def _():
            w_copy(w_hbm, wv_ref, w_sems, idx_ref, t + R).start()

    grid_spec = pltpu.PrefetchScalarGridSpec(
        num_scalar_prefetch=1,
        grid=(N_DEV,),
        in_specs=[
            pl.BlockSpec((m_total, k_shard), lambda t, idx: (0, 0)),
            pl.BlockSpec(memory_space=pl.ANY),
        ],
        out_specs=pl.BlockSpec((m_blk, n_total), lambda t, idx: (0, 0)),
        scratch_shapes=[
            pltpu.VMEM((m_total, k_shard), jnp.bfloat16),
            pltpu.VMEM((N_DEV, m_blk, k_shard), jnp.bfloat16),
            pltpu.VMEM((R, k_shard, n_total), jnp.float32),
            pltpu.SemaphoreType.DMA((N_DEV,)),
            pltpu.SemaphoreType.DMA((N_DEV,)),
            pltpu.SemaphoreType.DMA((R,)),
        ],
    )

    idx = jnp.mod(
        lax.axis_index("i") + jnp.arange(N_DEV, dtype=jnp.int32), N_DEV
    ).astype(jnp.int32)

    return pl.pallas_call(
        body,
        grid_spec=grid_spec,
        out_shape=jax.ShapeDtypeStruct((m_blk, n_total), jnp.float32),
        compiler_params=pltpu.CompilerParams(
            dimension_semantics=("arbitrary",),
            vmem_limit_bytes=60 * 1024 * 1024,
            collective_id=0,
        ),
    )(idx, x, w_mat)
